# baseline (speedup 1.0000x reference)
import functools

import jax
import jax.numpy as jnp
from jax import lax
from jax.experimental import pallas as pl
from jax.experimental.pallas import tpu as pltpu
from jax.experimental.pallas import tpu_sc as plsc

VOCAB = 1000000
D = 64
DP = 128
NNEG = 5
B = 16384
L = 16
NC = 1
NS = 16
NW = NC * NS
EPW = B // NW
SUB = 64
NSUB = EPW // SUB
SCORES = 1 + NNEG
NCOMBO = 4
QW = NCOMBO * L


def _sc_gather_dot(iword, owords, nwords, ivw2, ovw2):
  mesh = plsc.VectorSubcoreMesh(
      core_axis_name="c", subcore_axis_name="s",
      num_cores=NC, num_subcores=NS)

  @functools.partial(
      pl.kernel,
      out_type=jax.ShapeDtypeStruct((B * SCORES * QW,), jnp.float32),
      mesh=mesh,
      scratch_types=[
          pltpu.VMEM((SUB,), jnp.int32),
          pltpu.VMEM((SUB,), jnp.int32),
          pltpu.VMEM((SUB * NNEG,), jnp.int32),
          pltpu.VMEM((SUB, DP), jnp.float32),
          pltpu.VMEM((SUB, DP), jnp.float32),
          pltpu.VMEM((SUB * NNEG, DP), jnp.float32),
          pltpu.VMEM((SUB * SCORES * QW,), jnp.float32),
          pltpu.SemaphoreType.DMA,
      ],
  )
  def k(iw_hbm, ow_hbm, nw_hbm, ivw_hbm, ovw_hbm, out_hbm,
        iw_idx, ow_idx, nw_idx, iv_rows, ov_rows, nv_rows, out_buf, sem):
    wid = lax.axis_index("s") * NC + lax.axis_index("c")

    def subchunk(c, carry):
      off = wid * EPW + c * SUB
      pltpu.sync_copy(iw_hbm.at[pl.ds(off, SUB)], iw_idx)
      pltpu.sync_copy(ow_hbm.at[pl.ds(off, SUB)], ow_idx)
      pltpu.sync_copy(nw_hbm.at[pl.ds(off * NNEG, SUB * NNEG)], nw_idx)
      for v in range(SUB // L):
        s = pl.ds(v * L, L)
        iw_idx[s] = lax.shift_right_logical(iw_idx[s], 1)
        ow_idx[s] = lax.shift_right_logical(ow_idx[s], 1)
      for v in range(SUB * NNEG // L):
        s = pl.ds(v * L, L)
        nw_idx[s] = lax.shift_right_logical(nw_idx[s], 1)
      h_iv = pltpu.async_copy(ivw_hbm.at[iw_idx], iv_rows, sem)
      h_ov = pltpu.async_copy(ovw_hbm.at[ow_idx], ov_rows, sem)
      h_nv = [
          pltpu.async_copy(ovw_hbm.at[nw_idx.at[pl.ds(j * SUB, SUB)]],
                           nv_rows.at[pl.ds(j * SUB, SUB)], sem)
          for j in range(NNEG)
      ]
      h_iv.wait()
      h_ov.wait()
      for h in h_nv:
        h.wait()

      def elem(b, carry2):
        iv = [[iv_rows[b, pl.ds(x * D + L * j, L)] for j in range(D // L)]
              for x in range(2)]

        def score_partials(rows_ref, r, qbase):
          for y in range(2):
            t = [rows_ref[r, pl.ds(y * D + L * j, L)] for j in range(D // L)]
            for x in range(2):
              acc = iv[x][0] * t[0]
              for j in range(1, D // L):
                acc += iv[x][j] * t[j]
              out_buf[pl.ds(qbase + (x * 2 + y) * L, L)] = acc

        score_partials(ov_rows, b, b * SCORES * QW)
        for k2 in range(NNEG):
          score_partials(nv_rows, b * NNEG + k2,
                         (b * SCORES + 1 + k2) * QW)
        return carry2

      lax.fori_loop(0, SUB, elem, 0)
      pltpu.sync_copy(out_buf,
                      out_hbm.at[pl.ds(off * SCORES * QW, SUB * SCORES * QW)])
      return carry

    lax.fori_loop(0, NSUB, subchunk, 0)

  return k(iword, owords, nwords, ivw2, ovw2)


def _tc_reduce_loss(partials2d, combo8):
  rows, lanes = partials2d.shape
  gpr = lanes // L
  blk = 6144
  nblk = rows // blk

  def body(x_ref, c_ref, o_ref):
    pid = pl.program_id(0)
    x = x_ref[...]
    cmb = c_ref[...]
    gi = lax.broadcasted_iota(jnp.int32, (lanes, gpr), 0)
    gj = lax.broadcasted_iota(jnp.int32, (lanes, gpr), 1)
    g = (gi // L == gj).astype(jnp.float32)
    s = jax.lax.dot(x, g, preferred_element_type=jnp.float32)
    ci = lax.broadcasted_iota(jnp.int32, (blk, gpr), 1)
    keep = (ci % NCOMBO) == cmb
    sk = jnp.where(keep, s, 0.0)
    hi = lax.broadcasted_iota(jnp.int32, (gpr, 2), 0)
    hj = lax.broadcasted_iota(jnp.int32, (gpr, 2), 1)
    h = (hi // NCOMBO == hj).astype(jnp.float32)
    z = jax.lax.dot(sk, h, preferred_element_type=jnp.float32)
    ri = pid * blk + lax.broadcasted_iota(jnp.int32, (blk, 2), 0)
    qi = ri * 2 + lax.broadcasted_iota(jnp.int32, (blk, 2), 1)
    pos = (qi % SCORES) == 0
    zs = jnp.where(pos, z, -z)
    loss = jnp.minimum(zs, 0.0) - jnp.log(1.0 + jnp.exp(-jnp.abs(zs)))
    part = jnp.full((1, 1), 0.0, jnp.float32) - jnp.sum(loss) / B

    @pl.when(pid == 0)
    def _():
      o_ref[...] = jnp.zeros((1, 1), jnp.float32)

    o_ref[...] += part

  return pl.pallas_call(
      body,
      grid=(nblk,),
      in_specs=[
          pl.BlockSpec((blk, lanes), lambda i: (i, 0)),
          pl.BlockSpec((blk, gpr), lambda i: (i, 0)),
      ],
      out_specs=pl.BlockSpec((1, 1), lambda i: (0, 0)),
      out_shape=jax.ShapeDtypeStruct((1, 1), jnp.float32),
  )(partials2d, combo8)


def kernel(iword, owords, ivec_weight, ovec_weight):
  iword = iword.astype(jnp.int32)
  owords = owords.astype(jnp.int32)
  nwords = jax.random.randint(
      jax.random.key(1), (B, NNEG), 0, VOCAB - 1).astype(jnp.int32)
  ivw2 = ivec_weight.reshape(VOCAB // 2, DP)
  ovw2 = ovec_weight.reshape(VOCAB // 2, DP)
  partials = _sc_gather_dot(iword, owords, nwords.reshape(B * NNEG),
                            ivw2, ovw2)
  partials2d = partials.reshape(B * SCORES * QW // 128, 128)
  tgt = jnp.concatenate([owords[:, None], nwords], axis=1)
  combos = (iword[:, None] & 1) * 2 + (tgt & 1)
  combo8 = jnp.repeat(combos.reshape(-1, 2), NCOMBO, axis=1)
  out = _tc_reduce_loss(partials2d, combo8)
  return out.reshape(())


# speedup vs baseline: 1.1431x; 1.1431x over previous
import functools

import jax
import jax.numpy as jnp
from jax import lax
from jax.experimental import pallas as pl
from jax.experimental.pallas import tpu as pltpu
from jax.experimental.pallas import tpu_sc as plsc

VOCAB = 1000000
D = 64
NNEG = 5
B = 16384
L = 16
NC = 2
NS = 16
NW = NC * NS
EPW = B // NW
SUB = 128
NSUB = EPW // SUB
SCORES = 1 + NNEG


def _sc_gather_dot(iword, owords, nwords, ivw, ovw):
  mesh = plsc.VectorSubcoreMesh(
      core_axis_name="c", subcore_axis_name="s",
      num_cores=NC, num_subcores=NS)

  @functools.partial(
      pl.kernel,
      out_type=jax.ShapeDtypeStruct((B * SCORES, L), jnp.float32),
      mesh=mesh,
      compiler_params=pltpu.CompilerParams(
          use_tc_tiling_on_sc=False, has_side_effects=False),
      scratch_types=[
          pltpu.VMEM((SUB,), jnp.int32),
          pltpu.VMEM((SUB,), jnp.int32),
          [pltpu.VMEM((SUB,), jnp.int32) for _ in range(NNEG)],
          pltpu.VMEM((SUB, D), jnp.float32),
          pltpu.VMEM((SUB, D), jnp.float32),
          pltpu.VMEM((SUB * NNEG, D), jnp.float32),
          pltpu.VMEM((SUB * SCORES, L), jnp.float32),
          pltpu.SemaphoreType.DMA,
      ],
  )
  def k(iw_hbm, ow_hbm, nw_hbm, ivw_hbm, ovw_hbm, out_hbm,
        iw_idx, ow_idx, nw_idx, iv_rows, ov_rows, nv_rows, out_buf, sem):
    wid = lax.axis_index("s") * NC + lax.axis_index("c")
    for c in range(NSUB):
      off = wid * EPW + c * SUB
      pltpu.sync_copy(iw_hbm.at[pl.ds(off, SUB)], iw_idx)
      pltpu.sync_copy(ow_hbm.at[pl.ds(off, SUB)], ow_idx)
      for j in range(NNEG):
        pltpu.sync_copy(nw_hbm.at[pl.ds(off * NNEG + j * SUB, SUB)], nw_idx[j])
      h_iv = pltpu.async_copy(ivw_hbm.at[iw_idx], iv_rows, sem)
      h_ov = pltpu.async_copy(ovw_hbm.at[ow_idx], ov_rows, sem)
      h_nv = [
          pltpu.async_copy(ovw_hbm.at[nw_idx[j]],
                           nv_rows.at[pl.ds(j * SUB, SUB)], sem)
          for j in range(NNEG)
      ]
      h_iv.wait()
      h_ov.wait()
      for h in h_nv:
        h.wait()

      def elem(b, carry):
        iv = [iv_rows[b, pl.ds(L * j, L)] for j in range(D // L)]
        acc = iv[0] * ov_rows[b, pl.ds(0, L)]
        for j in range(1, D // L):
          acc += iv[j] * ov_rows[b, pl.ds(L * j, L)]
        out_buf[b * SCORES, :] = acc
        for k2 in range(NNEG):
          r = b * NNEG + k2
          nacc = iv[0] * nv_rows[r, pl.ds(0, L)]
          for j in range(1, D // L):
            nacc += iv[j] * nv_rows[r, pl.ds(L * j, L)]
          out_buf[b * SCORES + 1 + k2, :] = nacc
        return carry

      lax.fori_loop(0, SUB, elem, 0)
      pltpu.sync_copy(out_buf, out_hbm.at[pl.ds(off * SCORES, SUB * SCORES)])

  return k(iword, owords, nwords, ivw, ovw)


def _tc_reduce_loss(partials2d):
  rows, lanes = partials2d.shape
  groups_per_row = lanes // L

  def body(x_ref, o_ref):
    x = x_ref[...]
    gi = lax.broadcasted_iota(jnp.int32, (lanes, groups_per_row), 0)
    gj = lax.broadcasted_iota(jnp.int32, (lanes, groups_per_row), 1)
    g = (gi // L == gj).astype(jnp.float32)
    s = jax.lax.dot(x, g, preferred_element_type=jnp.float32)
    ri = lax.broadcasted_iota(jnp.int32, (rows, groups_per_row), 0)
    ci = lax.broadcasted_iota(jnp.int32, (rows, groups_per_row), 1)
    gid = ri * groups_per_row + ci
    pos = (gid % SCORES) == 0
    z = jnp.where(pos, s, -s)
    loss = jnp.minimum(z, 0.0) - jnp.log(1.0 + jnp.exp(-jnp.abs(z)))
    o_ref[...] = jnp.full((1, 1), 0.0, jnp.float32) - jnp.sum(loss) / B

  return pl.pallas_call(
      body,
      out_shape=jax.ShapeDtypeStruct((1, 1), jnp.float32),
  )(partials2d)


def kernel(iword, owords, ivec_weight, ovec_weight):
  iword = iword.astype(jnp.int32)
  owords = owords.astype(jnp.int32)
  nwords = jax.random.randint(
      jax.random.key(1), (B, NNEG), 0, VOCAB - 1).astype(jnp.int32)
  partials = _sc_gather_dot(iword, owords, nwords.reshape(B * NNEG),
                            ivec_weight, ovec_weight)
  partials2d = partials.reshape(B * SCORES * L // 128, 128)
  out = _tc_reduce_loss(partials2d)
  return out.reshape(())
